# Initial kernel scaffold; baseline (speedup 1.0000x reference)
import jax
import jax.numpy as jnp
from jax.experimental import pallas as pl
from jax.experimental.pallas import tpu as pltpu
from jax.experimental.pallas import tpu_sc as plsc

_N, _D, _K = 4096, 256, 8192
_TM = 256
_W = 64


def _csq_kernel(cbt_ref, csq_ref):
    c = cbt_ref[...]
    csq_ref[...] = jnp.sum(c * c, axis=0, keepdims=True)


def _argmin_kernel(csq_ref, x_ref, cbt_ref, idx_ref):
    x = x_ref[...]
    x_sq = jnp.sum(x * x, axis=1, keepdims=True)
    cross = jax.lax.dot_general(
        x, cbt_ref[...], (((1,), (0,)), ((), ())),
        preferred_element_type=jnp.float32)
    d2 = jnp.maximum(x_sq + csq_ref[...] - 2.0 * cross, 0.0)
    dist = jnp.sqrt(jnp.maximum(d2, 1e-12))
    idx_ref[0, 0, :] = jnp.argmin(dist, axis=1).astype(jnp.int32)


def _argmin_indices(x, cbt):
    csq = pl.pallas_call(
        _csq_kernel,
        out_shape=jax.ShapeDtypeStruct((1, _K), jnp.float32),
    )(cbt)
    idx = pl.pallas_call(
        _argmin_kernel,
        grid=(_N // _TM,),
        in_specs=[
            pl.BlockSpec((1, _K), lambda i: (0, 0)),
            pl.BlockSpec((_TM, _D), lambda i: (i, 0)),
            pl.BlockSpec((_D, _K), lambda i: (0, 0)),
        ],
        out_specs=pl.BlockSpec((1, 1, _TM), lambda i: (i, 0, 0)),
        out_shape=jax.ShapeDtypeStruct((_N // _TM, 1, _TM), jnp.int32),
        compiler_params=pltpu.CompilerParams(
            dimension_semantics=("parallel",)),
    )(csq, x, cbt)
    return idx.reshape(_N)


def _sc_gather(table, indices):
    mesh = plsc.VectorSubcoreMesh(
        core_axis_name="core", subcore_axis_name="subcore")
    i2 = indices.reshape(1, _N)

    @pl.kernel(out_type=jax.ShapeDtypeStruct((_N, _D), table.dtype),
               mesh=mesh)
    def gk(tab_hbm, i_hbm, o_hbm):
        def body(i_vmem, o_vmem):
            pltpu.sync_copy(tab_hbm.at[i_vmem.at[0]], o_vmem)

        pltpu.emit_pipeline(
            body,
            grid=(_N // _W,),
            in_specs=[pl.BlockSpec((1, _W), lambda i: (0, i))],
            out_specs=[pl.BlockSpec((_W, _D), lambda i: (i, 0))],
            core_axis_name=("core", "subcore"),
            dimension_semantics=(pltpu.PARALLEL,),
        )(i_hbm, o_hbm)

    return gk(table, i2)


def kernel(x, codebook, embedding_table):
    cbt = codebook.T
    indices = _argmin_indices(x, cbt)
    return _sc_gather(embedding_table, indices)


# speedup vs baseline: 1.0317x; 1.0317x over previous
import jax
import jax.numpy as jnp
from jax.experimental import pallas as pl
from jax.experimental.pallas import tpu as pltpu
from jax.experimental.pallas import tpu_sc as plsc

_N, _D, _K = 4096, 256, 8192
_TM = 256
_W = 128


def _csq_kernel(cbt_ref, csq_ref):
    c = cbt_ref[...]
    csq_ref[...] = jnp.sum(c * c, axis=0, keepdims=True)


def _argmin_kernel(csq_ref, x_ref, cbt_ref, idx_ref):
    x = x_ref[...]
    x_sq = jnp.sum(x * x, axis=1, keepdims=True)
    cross = jax.lax.dot_general(
        x, cbt_ref[...], (((1,), (0,)), ((), ())),
        preferred_element_type=jnp.float32)
    d2 = jnp.maximum(x_sq + csq_ref[...] - 2.0 * cross, 0.0)
    dist = jnp.sqrt(jnp.maximum(d2, 1e-12))
    idx_ref[0, 0, :] = jnp.argmin(dist, axis=1).astype(jnp.int32)


def _argmin_indices(x, cbt):
    csq = pl.pallas_call(
        _csq_kernel,
        out_shape=jax.ShapeDtypeStruct((1, _K), jnp.float32),
    )(cbt)
    idx = pl.pallas_call(
        _argmin_kernel,
        grid=(_N // _TM,),
        in_specs=[
            pl.BlockSpec((1, _K), lambda i: (0, 0)),
            pl.BlockSpec((_TM, _D), lambda i: (i, 0)),
            pl.BlockSpec((_D, _K), lambda i: (0, 0)),
        ],
        out_specs=pl.BlockSpec((1, 1, _TM), lambda i: (i, 0, 0)),
        out_shape=jax.ShapeDtypeStruct((_N // _TM, 1, _TM), jnp.int32),
        compiler_params=pltpu.CompilerParams(
            dimension_semantics=("parallel",)),
    )(csq, x, cbt)
    return idx.reshape(_N)


def _sc_gather(table, indices):
    mesh = plsc.VectorSubcoreMesh(
        core_axis_name="core", subcore_axis_name="subcore")
    i2 = indices.reshape(1, _N)

    @pl.kernel(out_type=jax.ShapeDtypeStruct((_N, _D), table.dtype),
               mesh=mesh)
    def gk(tab_hbm, i_hbm, o_hbm):
        def body(i_vmem, o_vmem):
            pltpu.sync_copy(tab_hbm.at[i_vmem.at[0]], o_vmem)

        pltpu.emit_pipeline(
            body,
            grid=(_N // _W,),
            in_specs=[pl.BlockSpec((1, _W), lambda i: (0, i))],
            out_specs=[pl.BlockSpec((_W, _D), lambda i: (i, 0))],
            core_axis_name=("core", "subcore"),
            dimension_semantics=(pltpu.PARALLEL,),
        )(i_hbm, o_hbm)

    return gk(table, i2)


def kernel(x, codebook, embedding_table):
    cbt = codebook.T
    indices = _argmin_indices(x, cbt)
    return _sc_gather(embedding_table, indices)


# speedup vs baseline: 1.2997x; 1.2598x over previous
import jax
import jax.numpy as jnp
from jax.experimental import pallas as pl
from jax.experimental.pallas import tpu as pltpu
from jax.experimental.pallas import tpu_sc as plsc

_N, _D, _K = 4096, 256, 8192
_TM = 256
_W = 128
_FIX = 256
_MARGIN = 1.0 + 2.0 ** -18


def _csq_kernel(cbt_ref, csq_ref):
    c = cbt_ref[...]
    csq_ref[...] = jnp.sum(c * c, axis=0, keepdims=True)


def _cand_kernel(csq_ref, x_ref, cbt_ref, idx_ref):
    x = x_ref[...]
    x_sq = jnp.sum(x * x, axis=1, keepdims=True)
    nb = 4
    kb = _K // nb
    ts, ms = [], []
    for b in range(nb):
        sl = pl.ds(b * kb, kb)
        cross = jax.lax.dot_general(
            x, cbt_ref[:, sl], (((1,), (0,)), ((), ())),
            preferred_element_type=jnp.float32)
        t = x_sq + csq_ref[:, sl] - 2.0 * cross
        ts.append(t)
        ms.append(jnp.min(t, axis=1, keepdims=True))
    m = jnp.minimum(jnp.minimum(ms[0], ms[1]), jnp.minimum(ms[2], ms[3]))
    bthr = jnp.maximum(m, 1e-12) * _MARGIN
    acc = jnp.zeros((_TM, 1), jnp.int32)
    for b in range(nb):
        iota = jax.lax.broadcasted_iota(
            jnp.int32, (_TM, kb), 1) + (16384 + b * kb)
        acc = acc + jnp.sum(jnp.where(ts[b] <= bthr, iota, 0),
                            axis=1, keepdims=True)
    idx = jnp.where(acc < 32768, acc - 16384, -1)
    idx_ref[0, 0, :] = idx[:, 0]


def _fix_kernel(csq_ref, x_ref, cbt_ref, idx_ref):
    x = x_ref[...]
    x_sq = jnp.sum(x * x, axis=1, keepdims=True)
    cross = jax.lax.dot_general(
        x, cbt_ref[...], (((1,), (0,)), ((), ())),
        preferred_element_type=jnp.float32)
    t = x_sq + csq_ref[...] - 2.0 * cross
    dist = jnp.sqrt(jnp.maximum(t, 1e-12))
    idx_ref[0, 0, :] = jnp.argmin(dist, axis=1).astype(jnp.int32)


def _sc_gather(table, indices):
    mesh = plsc.VectorSubcoreMesh(
        core_axis_name="core", subcore_axis_name="subcore")
    i2 = indices.reshape(1, _N)

    @pl.kernel(out_type=jax.ShapeDtypeStruct((_N, _D), table.dtype),
               mesh=mesh)
    def gk(tab_hbm, i_hbm, o_hbm):
        def body(i_vmem, o_vmem):
            pltpu.sync_copy(tab_hbm.at[i_vmem.at[0]], o_vmem)

        pltpu.emit_pipeline(
            body,
            grid=(_N // _W,),
            in_specs=[pl.BlockSpec((1, _W), lambda i: (0, i))],
            out_specs=[pl.BlockSpec((_W, _D), lambda i: (i, 0))],
            core_axis_name=("core", "subcore"),
            dimension_semantics=(pltpu.PARALLEL,),
        )(i_hbm, o_hbm)

    return gk(table, i2)


def kernel(x, codebook, embedding_table):
    cbt = codebook.T
    csq = pl.pallas_call(
        _csq_kernel,
        out_shape=jax.ShapeDtypeStruct((1, _K), jnp.float32),
    )(cbt)

    j1 = pl.pallas_call(
        _cand_kernel,
        grid=(_N // _TM,),
        in_specs=[
            pl.BlockSpec((1, _K), lambda i: (0, 0)),
            pl.BlockSpec((_TM, _D), lambda i: (i, 0)),
            pl.BlockSpec((_D, _K), lambda i: (0, 0)),
        ],
        out_specs=pl.BlockSpec((1, 1, _TM), lambda i: (i, 0, 0)),
        out_shape=jax.ShapeDtypeStruct((_N // _TM, 1, _TM), jnp.int32),
        compiler_params=pltpu.CompilerParams(
            dimension_semantics=("parallel",)),
    )(csq, x, cbt).reshape(_N)

    flagged = j1 < 0

    fix_rows = jnp.where(flagged, size=_FIX, fill_value=0)[0]
    x_fix = x[fix_rows]
    fixed = pl.pallas_call(
        _fix_kernel,
        grid=(1,),
        in_specs=[
            pl.BlockSpec((1, _K), lambda i: (0, 0)),
            pl.BlockSpec((_FIX, _D), lambda i: (0, 0)),
            pl.BlockSpec((_D, _K), lambda i: (0, 0)),
        ],
        out_specs=pl.BlockSpec((1, 1, _FIX), lambda i: (0, 0, 0)),
        out_shape=jax.ShapeDtypeStruct((1, 1, _FIX), jnp.int32),
    )(csq, x_fix, cbt).reshape(_FIX)

    indices = j1.at[fix_rows].set(fixed)
    return _sc_gather(embedding_table, indices)


# speedup vs baseline: 1.3520x; 1.0403x over previous
import jax
import jax.numpy as jnp
from jax.experimental import pallas as pl
from jax.experimental.pallas import tpu as pltpu
from jax.experimental.pallas import tpu_sc as plsc

_N, _D, _K = 4096, 256, 8192
_TM = 256
_W = 128
_FIX = 256
_MARGIN = 1.0 + 2.0 ** -18
_NB = 4


def _cand_kernel(csq_ref, x_ref, cb_ref, iota_ref, idx_ref):
    x = x_ref[...]
    x_sq = jnp.sum(x * x, axis=1, keepdims=True)
    kb = _K // _NB
    ts, ms = [], []
    for b in range(_NB):
        sl = pl.ds(b * kb, kb)
        cross = jax.lax.dot_general(
            x, cb_ref[sl, :], (((1,), (1,)), ((), ())),
            preferred_element_type=jnp.float32)
        t = x_sq + csq_ref[:, sl] - 2.0 * cross
        ts.append(t)
        ms.append(jnp.min(t, axis=1, keepdims=True))
    m = jnp.minimum(jnp.minimum(ms[0], ms[1]), jnp.minimum(ms[2], ms[3]))
    bthr = jnp.maximum(m, 1e-12) * _MARGIN
    acc = jnp.zeros((_TM, 1), jnp.int32)
    for b in range(_NB):
        sl = pl.ds(b * kb, kb)
        iota = jnp.broadcast_to(iota_ref[:, sl], (_TM, kb))
        acc = acc + jnp.sum(jnp.where(ts[b] <= bthr, iota, 0),
                            axis=1, keepdims=True)
    idx = jnp.where(acc < 32768, acc - 16384, -1)
    idx_ref[0, 0, :] = idx[:, 0]


def _fix_kernel(csq_ref, xsq_ref, x_ref, cb_ref, idx_ref):
    cross = jax.lax.dot_general(
        x_ref[...], cb_ref[...], (((1,), (1,)), ((), ())),
        preferred_element_type=jnp.float32)
    t = xsq_ref[...] + csq_ref[...] - 2.0 * cross
    dist = jnp.sqrt(jnp.maximum(t, 1e-12))
    idx_ref[0, 0, :] = jnp.argmin(dist, axis=1).astype(jnp.int32)


def _sc_gather(table, indices):
    mesh = plsc.VectorSubcoreMesh(
        core_axis_name="core", subcore_axis_name="subcore")
    i2 = indices.reshape(1, _N)

    @pl.kernel(out_type=jax.ShapeDtypeStruct((_N, _D), table.dtype),
               mesh=mesh)
    def gk(tab_hbm, i_hbm, o_hbm):
        def body(i_vmem, o_vmem):
            pltpu.sync_copy(tab_hbm.at[i_vmem.at[0]], o_vmem)

        pltpu.emit_pipeline(
            body,
            grid=(_N // _W,),
            in_specs=[pl.BlockSpec((1, _W), lambda i: (0, i))],
            out_specs=[pl.BlockSpec((_W, _D), lambda i: (i, 0))],
            core_axis_name=("core", "subcore"),
            dimension_semantics=(pltpu.PARALLEL,),
        )(i_hbm, o_hbm)

    return gk(table, i2)


def kernel(x, codebook, embedding_table):
    csq = jnp.sum(codebook * codebook, axis=-1)[None, :]
    xsq = jnp.sum(x * x, axis=-1, keepdims=True)
    iota1 = (16384 + jnp.arange(_K, dtype=jnp.int32)).reshape(1, _K)

    j1 = pl.pallas_call(
        _cand_kernel,
        grid=(_N // _TM,),
        in_specs=[
            pl.BlockSpec((1, _K), lambda i: (0, 0)),
            pl.BlockSpec((_TM, _D), lambda i: (i, 0)),
            pl.BlockSpec((_K, _D), lambda i: (0, 0)),
            pl.BlockSpec((1, _K), lambda i: (0, 0)),
        ],
        out_specs=pl.BlockSpec((1, 1, _TM), lambda i: (i, 0, 0)),
        out_shape=jax.ShapeDtypeStruct((_N // _TM, 1, _TM), jnp.int32),
        compiler_params=pltpu.CompilerParams(
            dimension_semantics=("arbitrary",)),
    )(csq, x, codebook, iota1).reshape(_N)

    flagged = j1 < 0
    fix_rows = jnp.where(flagged, size=_FIX, fill_value=0)[0]
    x_fix = x[fix_rows]
    xsq_fix = xsq[fix_rows]
    fixed = pl.pallas_call(
        _fix_kernel,
        grid=(1,),
        in_specs=[
            pl.BlockSpec((1, _K), lambda i: (0, 0)),
            pl.BlockSpec((_FIX, 1), lambda i: (0, 0)),
            pl.BlockSpec((_FIX, _D), lambda i: (0, 0)),
            pl.BlockSpec((_K, _D), lambda i: (0, 0)),
        ],
        out_specs=pl.BlockSpec((1, 1, _FIX), lambda i: (0, 0, 0)),
        out_shape=jax.ShapeDtypeStruct((1, 1, _FIX), jnp.int32),
    )(csq, xsq_fix, x_fix, codebook).reshape(_FIX)

    indices = j1.at[fix_rows].set(fixed)
    return _sc_gather(embedding_table, indices)
